# baseline (speedup 1.0000x reference)
import functools

import jax
import jax.numpy as jnp
from jax import lax
from jax.experimental import pallas as pl
from jax.experimental.pallas import tpu as pltpu
from jax.experimental.pallas import tpu_sc as plsc

N = 10000
D = 128
CHUNK = 1600
LANES = 16
UNROLL = 4
RC_SHIFT = 14


def _matmul_pack_body(x_ref, w_ref, b_ref, adj_ref, hp_ref, rc_ref):
    hid = lax.dot_general(
        w_ref[...], x_ref[...],
        (((1,), (1,)), ((), ())),
        preferred_element_type=jnp.float32,
    ) + b_ref[...]
    top = hid[:D // 2]
    bot = hid[D // 2:]
    tb = lax.bitcast_convert_type(top.astype(jnp.bfloat16), jnp.uint16)
    bb = lax.bitcast_convert_type(bot.astype(jnp.bfloat16), jnp.uint16)
    hp_ref[...] = (bb.astype(jnp.int32) << 16) | tb.astype(jnp.int32)
    rc_ref[...] = (adj_ref[0:1, :] << RC_SHIFT) + adj_ref[1:2, :]


def _add_transpose_body(a_ref, b_ref, e_ref, o_ref):
    s = (a_ref[...] + b_ref[...]).astype(jnp.bfloat16)
    o_ref[...] = lax.dot_general(
        s, e_ref[...],
        (((0,), (0,)), ((), ())),
        preferred_element_type=jnp.float32,
    )


def _sc_body(hp, rc, vals, outT,
             hp0, hp1, hp2, hp3,
             a0, a1, a2, a3, a4, a5, a6, a7,
             kb0, vb0, kb1, vb1, sem0, sem1):
    hps = (hp0, hp1, hp2, hp3)
    accs = (a0, a1, a2, a3, a4, a5, a6, a7)
    bufs = ((kb0, vb0, sem0), (kb1, vb1, sem1))
    core = lax.axis_index("c")
    sub = lax.axis_index("s")

    E = rc.shape[0]
    half = E // 2
    e_base = core * half
    n_chunks = half // CHUNK

    def _issue(k, which):
        kb, vb, sem = bufs[which]
        e0 = e_base + k * CHUNK
        pltpu.async_copy(rc.at[pl.ds(e0, CHUNK)], kb, sem)
        pltpu.async_copy(vals.at[pl.ds(e0, CHUNK)], vb, sem)

    def _drain(which):
        kb, vb, sem = bufs[which]
        pltpu.make_async_copy(rc.at[pl.ds(0, CHUNK)], kb, sem).wait()
        pltpu.make_async_copy(vals.at[pl.ds(0, CHUNK)], vb, sem).wait()

    def _process(which):
        kb, vb, _ = bufs[which]

        @plsc.parallel_loop(0, CHUNK // LANES, unroll=UNROLL)
        def _group_body(i):
            base = i * LANES
            k16 = kb[pl.ds(base, LANES)]
            v16 = vb[pl.ds(base, LANES)]
            c16 = k16 & ((1 << RC_SHIFT) - 1)
            r16 = lax.shift_right_logical(k16, RC_SHIFT)
            for j in range(4):
                g = plsc.load_gather(hps[j], [c16])
                topf = plsc.bitcast(g << 16, jnp.float32)
                botf = plsc.bitcast(g & jnp.int32(-65536), jnp.float32)
                plsc.addupdate_scatter(accs[2 * j], [r16], topf * v16)
                plsc.addupdate_scatter(accs[2 * j + 1], [r16], botf * v16)

    _issue(0, 0)
    _issue(1, 1)

    for j in range(4):
        pltpu.sync_copy(hp.at[4 * sub + j], hps[j])

    zero = jnp.zeros((LANES,), jnp.float32)

    @plsc.parallel_loop(0, N // LANES, unroll=4)
    def _zero_body(i):
        base = i * LANES
        for d in range(8):
            accs[d][pl.ds(base, LANES)] = zero

    def _outer(k, _):
        ca = 2 * k
        for which in range(2):
            c_cur = ca + which
            _drain(which)
            _process(which)

            @pl.when(c_cur + 2 < n_chunks)
            def _():
                _issue(c_cur + 2, which)
        return _

    lax.fori_loop(0, n_chunks // 2, _outer, None)

    for j in range(4):
        pltpu.sync_copy(accs[2 * j], outT.at[core, 4 * sub + j])
        pltpu.sync_copy(accs[2 * j + 1], outT.at[core, D // 2 + 4 * sub + j])


def _make_sc_call():
    mesh = plsc.VectorSubcoreMesh(core_axis_name="c", subcore_axis_name="s")
    return functools.partial(
        pl.kernel,
        mesh=mesh,
        out_type=jax.ShapeDtypeStruct((2, D, N), jnp.float32),
        compiler_params=pltpu.CompilerParams(needs_layout_passes=False),
        scratch_types=(
            [pltpu.VMEM((N,), jnp.int32) for _ in range(4)]
            + [pltpu.VMEM((N,), jnp.float32) for _ in range(8)]
            + [pltpu.VMEM((CHUNK,), jnp.int32),
               pltpu.VMEM((CHUNK,), jnp.float32)] * 2
            + [pltpu.SemaphoreType.DMA, pltpu.SemaphoreType.DMA]
        ),
    )(_sc_body)


def kernel(x, adj_indices, adj_values, W, b):
    n, d_in = x.shape
    d_out = W.shape[0]
    e = adj_values.shape[0]

    hp, rc2 = pl.pallas_call(
        _matmul_pack_body,
        out_shape=(
            jax.ShapeDtypeStruct((d_out // 2, n), jnp.int32),
            jax.ShapeDtypeStruct((1, e), jnp.int32),
        ),
    )(x.astype(jnp.bfloat16), W.astype(jnp.bfloat16), b[:, None], adj_indices)

    sc_call = _make_sc_call()
    outT2 = sc_call(hp, rc2.reshape(e), adj_values)

    eye = jnp.eye(d_out, dtype=jnp.bfloat16)
    out = pl.pallas_call(
        _add_transpose_body,
        out_shape=jax.ShapeDtypeStruct((n, d_out), jnp.float32),
    )(outT2[0], outT2[1], eye)
    return out


# speedup vs baseline: 1.0154x; 1.0154x over previous
import functools

import jax
import jax.numpy as jnp
from jax import lax
from jax.experimental import pallas as pl
from jax.experimental.pallas import tpu as pltpu
from jax.experimental.pallas import tpu_sc as plsc

N = 10000
D = 128
CHUNK = 1600
LANES = 16
UNROLL = 4
RC_SHIFT = 14


def _matmul_pack_body(x_ref, w_ref, b_ref, adj_ref, hp_ref, rc_ref):
    hid = lax.dot_general(
        w_ref[...], x_ref[...],
        (((1,), (1,)), ((), ())),
        preferred_element_type=jnp.float32,
    ) + b_ref[...]
    top = hid[:D // 2]
    bot = hid[D // 2:]
    tb = lax.bitcast_convert_type(top.astype(jnp.bfloat16), jnp.uint16)
    bb = lax.bitcast_convert_type(bot.astype(jnp.bfloat16), jnp.uint16)
    hp_ref[...] = (bb.astype(jnp.int32) << 16) | tb.astype(jnp.int32)
    rc_ref[...] = (adj_ref[0:1, :] << RC_SHIFT) + adj_ref[1:2, :]


def _add_transpose_body(a_ref, b_ref, e_ref, o_ref):
    o_ref[...] = lax.dot_general(
        a_ref[...] + b_ref[...], e_ref[...],
        (((0,), (0,)), ((), ())),
        preferred_element_type=jnp.float32,
    )


def _sc_body(hp, rc, vals, outT,
             hp0, hp1, hp2, hp3,
             a0, a1, a2, a3, a4, a5, a6, a7,
             kb0, vb0, kb1, vb1, sem0, sem1):
    hps = (hp0, hp1, hp2, hp3)
    accs = (a0, a1, a2, a3, a4, a5, a6, a7)
    bufs = ((kb0, vb0, sem0), (kb1, vb1, sem1))
    core = lax.axis_index("c")
    sub = lax.axis_index("s")

    E = rc.shape[0]
    half = E // 2
    e_base = core * half
    n_chunks = half // CHUNK

    def _issue(k, which):
        kb, vb, sem = bufs[which]
        e0 = e_base + k * CHUNK
        pltpu.async_copy(rc.at[pl.ds(e0, CHUNK)], kb, sem)
        pltpu.async_copy(vals.at[pl.ds(e0, CHUNK)], vb, sem)

    def _drain(which):
        kb, vb, sem = bufs[which]
        pltpu.make_async_copy(rc.at[pl.ds(0, CHUNK)], kb, sem).wait()
        pltpu.make_async_copy(vals.at[pl.ds(0, CHUNK)], vb, sem).wait()

    def _process(which):
        kb, vb, _ = bufs[which]

        @plsc.parallel_loop(0, CHUNK // LANES, unroll=UNROLL)
        def _group_body(i):
            base = i * LANES
            k16 = kb[pl.ds(base, LANES)]
            v16 = vb[pl.ds(base, LANES)]
            c16 = k16 & ((1 << RC_SHIFT) - 1)
            r16 = lax.shift_right_logical(k16, RC_SHIFT)
            for j in range(4):
                g = plsc.load_gather(hps[j], [c16])
                topf = plsc.bitcast(g << 16, jnp.float32)
                botf = plsc.bitcast(g & jnp.int32(-65536), jnp.float32)
                plsc.addupdate_scatter(accs[2 * j], [r16], topf * v16)
                plsc.addupdate_scatter(accs[2 * j + 1], [r16], botf * v16)

    _issue(0, 0)
    _issue(1, 1)

    for j in range(4):
        pltpu.sync_copy(hp.at[4 * sub + j], hps[j])

    zero = jnp.zeros((LANES,), jnp.float32)

    @plsc.parallel_loop(0, N // LANES, unroll=4)
    def _zero_body(i):
        base = i * LANES
        for d in range(8):
            accs[d][pl.ds(base, LANES)] = zero

    def _outer(k, _):
        ca = 2 * k
        for which in range(2):
            c_cur = ca + which
            _drain(which)
            _process(which)

            @pl.when(c_cur + 2 < n_chunks)
            def _():
                _issue(c_cur + 2, which)
        return _

    lax.fori_loop(0, n_chunks // 2, _outer, None)

    for j in range(4):
        pltpu.sync_copy(accs[2 * j], outT.at[core, 4 * sub + j])
        pltpu.sync_copy(accs[2 * j + 1], outT.at[core, D // 2 + 4 * sub + j])


def _make_sc_call():
    mesh = plsc.VectorSubcoreMesh(core_axis_name="c", subcore_axis_name="s")
    return functools.partial(
        pl.kernel,
        mesh=mesh,
        out_type=jax.ShapeDtypeStruct((2, D, N), jnp.float32),
        compiler_params=pltpu.CompilerParams(needs_layout_passes=False),
        scratch_types=(
            [pltpu.VMEM((N,), jnp.int32) for _ in range(4)]
            + [pltpu.VMEM((N,), jnp.float32) for _ in range(8)]
            + [pltpu.VMEM((CHUNK,), jnp.int32),
               pltpu.VMEM((CHUNK,), jnp.float32)] * 2
            + [pltpu.SemaphoreType.DMA, pltpu.SemaphoreType.DMA]
        ),
    )(_sc_body)


def kernel(x, adj_indices, adj_values, W, b):
    n, d_in = x.shape
    d_out = W.shape[0]
    e = adj_values.shape[0]

    hp, rc2 = pl.pallas_call(
        _matmul_pack_body,
        out_shape=(
            jax.ShapeDtypeStruct((d_out // 2, n), jnp.int32),
            jax.ShapeDtypeStruct((1, e), jnp.int32),
        ),
    )(x, W, b[:, None], adj_indices)

    sc_call = _make_sc_call()
    outT2 = sc_call(hp, rc2.reshape(e), adj_values)

    eye = jnp.eye(d_out, dtype=jnp.float32)
    out = pl.pallas_call(
        _add_transpose_body,
        out_shape=jax.ShapeDtypeStruct((n, d_out), jnp.float32),
    )(outT2[0], outT2[1], eye)
    return out
